# baseline (speedup 1.0000x reference)
import jax
import jax.numpy as jnp
from jax.experimental import pallas as pl
from jax.experimental.pallas import tpu as pltpu

_TOKENS = 16384
_HIDDEN = 2048
_E = 16
_BLK_T = 1024


def _router_body(x_ref, w_ref, b_ref, logits_ref, wts_ref, idx_ref, mask_ref):
    x = x_ref[...]
    w = w_ref[...]
    b = b_ref[...]
    logits = jax.lax.dot_general(
        x, w, (((1,), (1,)), ((), ())),
        preferred_element_type=jnp.float32) + b
    logits_ref[...] = logits

    m = jnp.max(logits, axis=1, keepdims=True)
    e = jnp.exp(logits - m)
    p = e / jnp.sum(e, axis=1, keepdims=True)

    iota = jax.lax.broadcasted_iota(jnp.int32, p.shape, 1)
    p1 = jnp.max(p, axis=1, keepdims=True)
    i1 = jnp.min(jnp.where(p == p1, iota, _E), axis=1, keepdims=True)
    oh1 = (iota == i1)
    pm = jnp.where(oh1, -1.0, p)
    p2 = jnp.max(pm, axis=1, keepdims=True)
    i2 = jnp.min(jnp.where(pm == p2, iota, _E), axis=1, keepdims=True)
    oh2 = (iota == i2)

    s = p1 + p2
    wts_ref[...] = jnp.concatenate([p1 / s, p2 / s], axis=1)
    idx_ref[...] = jnp.concatenate([i1, i2], axis=1)
    mask_ref[:, 0, :] = oh1.astype(jnp.int32).T
    mask_ref[:, 1, :] = oh2.astype(jnp.int32).T


def kernel(x, gate_w, gate_b):
    b2d = gate_b.reshape(1, _E)
    grid = (_TOKENS // _BLK_T,)
    logits, wts, idx, mask = pl.pallas_call(
        _router_body,
        grid=grid,
        in_specs=[
            pl.BlockSpec((_BLK_T, _HIDDEN), lambda i: (i, 0)),
            pl.BlockSpec((_E, _HIDDEN), lambda i: (0, 0)),
            pl.BlockSpec((1, _E), lambda i: (0, 0)),
        ],
        out_specs=[
            pl.BlockSpec((_BLK_T, _E), lambda i: (i, 0)),
            pl.BlockSpec((_BLK_T, 2), lambda i: (i, 0)),
            pl.BlockSpec((_BLK_T, 2), lambda i: (i, 0)),
            pl.BlockSpec((_E, 2, _BLK_T), lambda i: (0, 0, i)),
        ],
        out_shape=[
            jax.ShapeDtypeStruct((_TOKENS, _E), jnp.float32),
            jax.ShapeDtypeStruct((_TOKENS, 2), jnp.float32),
            jax.ShapeDtypeStruct((_TOKENS, 2), jnp.int32),
            jax.ShapeDtypeStruct((_E, 2, _TOKENS), jnp.int32),
        ],
    )(x, gate_w, b2d)
    return (logits, wts, idx, mask)
